# baseline (speedup 1.0000x reference)
import functools

import jax
import jax.numpy as jnp
from jax import lax
from jax.experimental import pallas as pl
from jax.experimental.pallas import tpu as pltpu
from jax.experimental.pallas import tpu_sc as plsc

EMB = 128
LN_EPS = 1e-5

NC = 2
NS = 16
NW = NC * NS

N_TOKENS = 4096 * 200
TOK_PER_W = N_TOKENS // NW
IDROWS_W = TOK_PER_W // 128
CHUNK = 256
GPC = CHUNK // 128
NCHUNK = TOK_PER_W // CHUNK
NB = 3


def _rsqrt(x):
    i = lax.bitcast_convert_type(x, jnp.int32)
    i = jnp.int32(0x5F3759DF) - (i >> 1)
    y = lax.bitcast_convert_type(i, jnp.float32)
    for _ in range(3):
        y = y * (1.5 - 0.5 * x * y * y)
    return y


@functools.partial(
    pl.kernel,
    mesh=plsc.VectorSubcoreMesh(core_axis_name="c", subcore_axis_name="s"),
    out_type=jax.ShapeDtypeStruct((N_TOKENS, EMB), jnp.float32),
    scratch_types=[
        pltpu.VMEM((IDROWS_W, 128), jnp.int32),
        pltpu.VMEM((NB, CHUNK, EMB), jnp.float32),
        pltpu.VMEM((2, EMB), jnp.float32),
        pltpu.SemaphoreType.DMA((NB,)),
        pltpu.SemaphoreType.DMA((NB,)),
    ],
    compiler_params=pltpu.CompilerParams(needs_layout_passes=False),
)
def _emb_ln(ids_hbm, table_hbm, w_hbm, b_hbm, out_hbm,
            idx_v, rows_v, wb_v, sg, so):
    wid = lax.axis_index("s") * NC + lax.axis_index("c")
    base_w = wid * TOK_PER_W

    pltpu.sync_copy(w_hbm, wb_v.at[0])
    pltpu.sync_copy(b_hbm, wb_v.at[1])
    w_vec = [wb_v[0, pl.ds(16 * j, 16)] for j in range(8)]
    b_vec = [wb_v[1, pl.ds(16 * j, 16)] for j in range(8)]

    pltpu.sync_copy(ids_hbm.at[pl.ds(wid * IDROWS_W, IDROWS_W)], idx_v)

    def fire_gathers(g, b):
        for u in range(GPC):
            pltpu.async_copy(
                table_hbm.at[idx_v.at[g * GPC + u]],
                rows_v.at[b].at[pl.ds(u * 128, 128)],
                sg.at[b],
            )

    def wait_gathers(b):
        for _ in range(GPC):
            pltpu.make_async_copy(
                table_hbm.at[idx_v.at[0]],
                rows_v.at[b].at[pl.ds(0, 128)],
                sg.at[b],
            ).wait()

    def fire_writeback(g, b):
        pltpu.async_copy(
            rows_v.at[b],
            out_hbm.at[pl.ds(base_w + g * CHUNK, CHUNK)],
            so.at[b],
        )

    def wait_writeback(b):
        pltpu.make_async_copy(
            rows_v.at[b],
            out_hbm.at[pl.ds(base_w, CHUNK)],
            so.at[b],
        ).wait()

    def compute_chunk(b):
        rbuf = rows_v.at[b]

        def group(t, _):
            row0 = t * 16
            rows = row0 + lax.iota(jnp.int32, 16)
            NA = 8
            s1 = [jnp.zeros((16,), jnp.float32) for _ in range(NA)]
            s2 = [jnp.zeros((16,), jnp.float32) for _ in range(NA)]
            for j in range(EMB):
                cj = jnp.full((16,), j, jnp.int32)
                x = plsc.load_gather(rbuf, [rows, cj])
                a = j % NA
                s1[a] = s1[a] + x
                s2[a] = s2[a] + x * x
            for h in (4, 2, 1):
                for a in range(h):
                    s1[a] = s1[a] + s1[a + h]
                    s2[a] = s2[a] + s2[a + h]
            mean = s1[0] * (1.0 / EMB)
            var = s2[0] * (1.0 / EMB) - mean * mean
            inv = _rsqrt(var + LN_EPS)
            for u in range(16):
                mu = mean[u]
                ivw = [inv[u] * w_vec[j] for j in range(8)]
                row = row0 + u
                for j in range(8):
                    x = rbuf[row, pl.ds(16 * j, 16)]
                    rbuf[row, pl.ds(16 * j, 16)] = (x - mu) * ivw[j] + b_vec[j]
            return 0

        lax.fori_loop(0, CHUNK // 16, group, 0)

    def step_fire(g, _):
        b = lax.rem(g, NB)
        bn = lax.rem(g + 2, NB)
        wait_gathers(b)
        compute_chunk(b)
        fire_writeback(g, b)

        @pl.when(g > 0)
        def _():
            wait_writeback(bn)

        fire_gathers(g + 2, bn)
        return 0

    def step_tail(g, _):
        b = lax.rem(g, NB)
        wait_gathers(b)
        compute_chunk(b)
        fire_writeback(g, b)
        return 0

    fire_gathers(0, 0)
    fire_gathers(1, 1)
    lax.fori_loop(0, NCHUNK - 2, step_fire, 0)
    lax.fori_loop(NCHUNK - 2, NCHUNK, step_tail, 0)
    for b in range(NB):
        wait_writeback(b)


def kernel(input_ids, table, ln_weight, ln_bias):
    ids = input_ids.reshape(-1).astype(jnp.int32).reshape(N_TOKENS // 128, 128)
    out = _emb_ln(ids, table, ln_weight, ln_bias)
    return out.reshape(input_ids.shape[0], input_ids.shape[1], EMB)


# speedup vs baseline: 1.9286x; 1.9286x over previous
import functools

import jax
import jax.numpy as jnp
from jax import lax
from jax.experimental import pallas as pl
from jax.experimental.pallas import tpu as pltpu
from jax.experimental.pallas import tpu_sc as plsc

EMB = 128
LN_EPS = 1e-5

NC = 2
NS = 16
NW = NC * NS

N_TOKENS = 4096 * 200
TOK_PER_W = N_TOKENS // NW
IDROWS_W = TOK_PER_W // 128
CHUNK = 256
GPC = CHUNK // 128
NCHUNK = TOK_PER_W // CHUNK
NB = 3


def _rsqrt(x):
    i = lax.bitcast_convert_type(x, jnp.int32)
    i = jnp.int32(0x5F3759DF) - (i >> 1)
    y = lax.bitcast_convert_type(i, jnp.float32)
    for _ in range(3):
        y = y * (1.5 - 0.5 * x * y * y)
    return y


@functools.partial(
    pl.kernel,
    mesh=plsc.VectorSubcoreMesh(core_axis_name="c", subcore_axis_name="s"),
    out_type=jax.ShapeDtypeStruct((N_TOKENS, EMB), jnp.float32),
    scratch_types=[
        pltpu.VMEM((IDROWS_W, 128), jnp.int32),
        pltpu.VMEM((NB, CHUNK, EMB), jnp.float32),
        pltpu.VMEM((2, EMB), jnp.float32),
        pltpu.SemaphoreType.DMA((NB,)),
        pltpu.SemaphoreType.DMA((NB,)),
    ],
    compiler_params=pltpu.CompilerParams(needs_layout_passes=False),
)
def _emb_ln(ids_hbm, table_hbm, w_hbm, b_hbm, out_hbm,
            idx_v, rows_v, wb_v, sg, so):
    wid = lax.axis_index("s") * NC + lax.axis_index("c")
    base_w = wid * TOK_PER_W

    pltpu.sync_copy(w_hbm, wb_v.at[0])
    pltpu.sync_copy(b_hbm, wb_v.at[1])
    w_vec = [wb_v[0, pl.ds(16 * j, 16)] for j in range(8)]
    b_vec = [wb_v[1, pl.ds(16 * j, 16)] for j in range(8)]

    pltpu.sync_copy(ids_hbm.at[pl.ds(wid * IDROWS_W, IDROWS_W)], idx_v)

    def fire_gathers(g, b):
        for u in range(GPC):
            pltpu.async_copy(
                table_hbm.at[idx_v.at[g * GPC + u]],
                rows_v.at[b].at[pl.ds(u * 128, 128)],
                sg.at[b],
            )

    def wait_gathers(b):
        for _ in range(GPC):
            pltpu.make_async_copy(
                table_hbm.at[idx_v.at[0]],
                rows_v.at[b].at[pl.ds(0, 128)],
                sg.at[b],
            ).wait()

    def fire_writeback(g, b):
        pltpu.async_copy(
            rows_v.at[b],
            out_hbm.at[pl.ds(base_w + g * CHUNK, CHUNK)],
            so.at[b],
        )

    def wait_writeback(b):
        pltpu.make_async_copy(
            rows_v.at[b],
            out_hbm.at[pl.ds(base_w, CHUNK)],
            so.at[b],
        ).wait()

    def compute_chunk(b):
        rbuf = rows_v.at[b]

        UNROLL = 8

        def tok(t, _):
            for u in range(UNROLL):
                row = t * UNROLL + u
                xs = [rbuf[row, pl.ds(16 * j, 16)] for j in range(8)]
                s1 = xs[0]
                s2 = xs[0] * xs[0]
                for j in range(1, 8):
                    s1 = s1 + xs[j]
                    s2 = s2 + xs[j] * xs[j]
                tot = plsc.cumsum(s1)[15]
                tot2 = plsc.cumsum(s2)[15]
                mean = tot * (1.0 / EMB)
                var = tot2 * (1.0 / EMB) - mean * mean
                inv = _rsqrt(var + LN_EPS)
                for j in range(8):
                    rbuf[row, pl.ds(16 * j, 16)] = (
                        (xs[j] - mean) * inv * w_vec[j] + b_vec[j]
                    )
            return 0

        lax.fori_loop(0, CHUNK // UNROLL, tok, 0)

    def step_fire(g, _):
        b = lax.rem(g, NB)
        bn = lax.rem(g + 2, NB)
        wait_gathers(b)
        compute_chunk(b)
        fire_writeback(g, b)

        @pl.when(g > 0)
        def _():
            wait_writeback(bn)

        fire_gathers(g + 2, bn)
        return 0

    def step_tail(g, _):
        b = lax.rem(g, NB)
        wait_gathers(b)
        compute_chunk(b)
        fire_writeback(g, b)
        return 0

    fire_gathers(0, 0)
    fire_gathers(1, 1)
    lax.fori_loop(0, NCHUNK - 2, step_fire, 0)
    lax.fori_loop(NCHUNK - 2, NCHUNK, step_tail, 0)
    for b in range(NB):
        wait_writeback(b)


def kernel(input_ids, table, ln_weight, ln_bias):
    ids = input_ids.reshape(-1).astype(jnp.int32).reshape(N_TOKENS // 128, 128)
    out = _emb_ln(ids, table, ln_weight, ln_bias)
    return out.reshape(input_ids.shape[0], input_ids.shape[1], EMB)


# speedup vs baseline: 3.6360x; 1.8853x over previous
import functools

import jax
import jax.numpy as jnp
from jax import lax
from jax.experimental import pallas as pl
from jax.experimental.pallas import tpu as pltpu
from jax.experimental.pallas import tpu_sc as plsc

EMB = 128
LN_EPS = 1e-5

NC = 2
NS = 16
NW = NC * NS

N_TOKENS = 4096 * 200
TOK_PER_W = N_TOKENS // NW
IDROWS_W = TOK_PER_W // 128
CHUNK = 256
GPC = CHUNK // 128
NCHUNK = TOK_PER_W // CHUNK
NB = 3


@functools.partial(
    pl.kernel,
    mesh=plsc.VectorSubcoreMesh(core_axis_name="c", subcore_axis_name="s"),
    out_type=jax.ShapeDtypeStruct((N_TOKENS, EMB), jnp.float32),
    scratch_types=[
        pltpu.VMEM((NB, GPC, 128), jnp.int32),
        pltpu.VMEM((NB, CHUNK, EMB), jnp.float32),
        pltpu.VMEM((2, EMB), jnp.float32),
        pltpu.VMEM((2, 64, 16), jnp.float32),
        pltpu.VMEM((2, CHUNK), jnp.float32),
        pltpu.SemaphoreType.DMA,
        pltpu.SemaphoreType.DMA,
        pltpu.SemaphoreType.DMA,
        pltpu.SemaphoreType.DMA,
        pltpu.SemaphoreType.DMA,
        pltpu.SemaphoreType.DMA,
        pltpu.SemaphoreType.DMA,
        pltpu.SemaphoreType.DMA,
        pltpu.SemaphoreType.DMA,
    ],
    compiler_params=pltpu.CompilerParams(needs_layout_passes=False),
)
def _emb_ln(ids_hbm, table_hbm, w_hbm, b_hbm, out_hbm,
            idx_v, rows_v, wb_v, cs_v, st_v,
            sg0, sg1, sg2, so0, so1, so2, si0, si1, si2):
    sg = [sg0, sg1, sg2]
    so = [so0, so1, so2]
    si = [si0, si1, si2]
    wid = lax.axis_index("s") * NC + lax.axis_index("c")
    base_w = wid * TOK_PER_W

    pltpu.sync_copy(w_hbm, wb_v.at[0])
    pltpu.sync_copy(b_hbm, wb_v.at[1])
    w_vec = [wb_v[0, pl.ds(16 * j, 16)] for j in range(8)]
    b_vec = [wb_v[1, pl.ds(16 * j, 16)] for j in range(8)]

    def fire_idx(g, b):
        pltpu.async_copy(
            ids_hbm.at[pl.ds(wid * IDROWS_W + g * GPC, GPC)],
            idx_v.at[b],
            si[b],
        )

    def wait_idx(b):
        pltpu.make_async_copy(
            ids_hbm.at[pl.ds(0, GPC)],
            idx_v.at[b],
            si[b],
        ).wait()

    def fire_gathers(g, b):
        for u in range(GPC):
            pltpu.async_copy(
                table_hbm.at[idx_v.at[b].at[u]],
                rows_v.at[b].at[pl.ds(u * 128, 128)],
                sg[b],
            )

    def wait_gathers(b):
        for _ in range(GPC):
            pltpu.make_async_copy(
                table_hbm.at[idx_v.at[b].at[0]],
                rows_v.at[b].at[pl.ds(0, 128)],
                sg[b],
            ).wait()

    def fire_writeback(g, b):
        pltpu.async_copy(
            rows_v.at[b],
            out_hbm.at[pl.ds(base_w + g * CHUNK, CHUNK)],
            so[b],
        )

    def wait_writeback(b):
        pltpu.make_async_copy(
            rows_v.at[b],
            out_hbm.at[pl.ds(base_w, CHUNK)],
            so[b],
        ).wait()

    def compute_chunk(b):
        rbuf = rows_v.at[b]
        UA = 4
        UC = 4

        def sub(k, _):
            base = k * 64

            def stats(t, _):
                for u in range(UA):
                    rm = t * UA + u
                    row = base + rm
                    xs = [rbuf[row, pl.ds(16 * j, 16)] for j in range(8)]
                    s1 = ((xs[0] + xs[1]) + (xs[2] + xs[3])) + (
                        (xs[4] + xs[5]) + (xs[6] + xs[7]))
                    sq = [x * x for x in xs]
                    s2 = ((sq[0] + sq[1]) + (sq[2] + sq[3])) + (
                        (sq[4] + sq[5]) + (sq[6] + sq[7]))
                    cs_v[0, rm, pl.ds(0, 16)] = plsc.cumsum(s1)
                    cs_v[1, rm, pl.ds(0, 16)] = plsc.cumsum(s2)
                return 0

            lax.fori_loop(0, 64 // UA, stats, 0)

            def moments(t, _):
                rows = t * 16 + lax.iota(jnp.int32, 16)
                lane15 = jnp.full((16,), 15, jnp.int32)
                tot = plsc.load_gather(cs_v.at[0], [rows, lane15])
                tot2 = plsc.load_gather(cs_v.at[1], [rows, lane15])
                mean = tot * (1.0 / EMB)
                var = tot2 * (1.0 / EMB) - mean * mean
                x = var + LN_EPS
                i = lax.bitcast_convert_type(x, jnp.int32)
                i = jnp.int32(0x5F3759DF) - (i >> 1)
                y = lax.bitcast_convert_type(i, jnp.float32)
                for _ in range(3):
                    y = y * (1.5 - 0.5 * x * y * y)
                sl = pl.ds(base + t * 16, 16)
                st_v[0, sl] = mean
                st_v[1, sl] = y
                return 0

            lax.fori_loop(0, 64 // 16, moments, 0)
            return 0

        lax.fori_loop(0, CHUNK // 64, sub, 0)

        def norm(t, _):
            mv = st_v[0, pl.ds(t * 16, 16)]
            vv = st_v[1, pl.ds(t * 16, 16)]
            for u in range(16):
                row = t * 16 + u
                mu = mv[u]
                iv = vv[u]
                for j in range(8):
                    x = rbuf[row, pl.ds(16 * j, 16)]
                    rbuf[row, pl.ds(16 * j, 16)] = (
                        (x - mu) * (iv * w_vec[j]) + b_vec[j]
                    )
            return 0

        lax.fori_loop(0, CHUNK // 16, norm, 0)

    def step(g, b, fire, wait_out, fire_ids=True):
        wait_gathers(b)
        if fire_ids:
            fire_idx(g + 3, b)
        compute_chunk(b)
        fire_writeback(g, b)
        if fire:
            bn = (b + 2) % NB
            wait_idx(bn)
            if wait_out:
                wait_writeback(bn)
            fire_gathers(g + 2, bn)

    pltpu.sync_copy(ids_hbm.at[pl.ds(wid * IDROWS_W, GPC)], idx_v.at[0])
    pltpu.sync_copy(ids_hbm.at[pl.ds(wid * IDROWS_W + GPC, GPC)], idx_v.at[1])
    fire_idx(2, 2)
    fire_gathers(0, 0)
    fire_gathers(1, 1)
    step(0, 0, fire=True, wait_out=False)

    def steady(k, _):
        for j in range(NB):
            g = 1 + k * NB + j
            step(g, (1 + j) % NB, fire=True, wait_out=True)
        return 0

    lax.fori_loop(0, (NCHUNK - 4) // NB, steady, 0)

    step(NCHUNK - 3, (NCHUNK - 3) % NB, fire=True, wait_out=True, fire_ids=False)
    step(NCHUNK - 2, (NCHUNK - 2) % NB, fire=False, wait_out=False, fire_ids=False)
    step(NCHUNK - 1, (NCHUNK - 1) % NB, fire=False, wait_out=False, fire_ids=False)
    for b in range(NB):
        wait_writeback(b)


def kernel(input_ids, table, ln_weight, ln_bias):
    ids = input_ids.reshape(-1).astype(jnp.int32).reshape(N_TOKENS // 128, 128)
    out = _emb_ln(ids, table, ln_weight, ln_bias)
    return out.reshape(input_ids.shape[0], input_ids.shape[1], EMB)


# speedup vs baseline: 5.6941x; 1.5660x over previous
import functools

import jax
import jax.numpy as jnp
from jax import lax
from jax.experimental import pallas as pl
from jax.experimental.pallas import tpu as pltpu
from jax.experimental.pallas import tpu_sc as plsc

EMB = 128
LN_EPS = 1e-5

NC = 2
NS = 16
NW = NC * NS

N_TOKENS = 4096 * 200
TOK_PER_W = N_TOKENS // NW
IDROWS_W = TOK_PER_W // 128
CHUNK = 256
GPC = CHUNK // 128
NCHUNK = TOK_PER_W // CHUNK
NB = 3


@functools.partial(
    pl.kernel,
    mesh=plsc.VectorSubcoreMesh(core_axis_name="c", subcore_axis_name="s"),
    out_type=jax.ShapeDtypeStruct((N_TOKENS, EMB), jnp.float32),
    scratch_types=[
        pltpu.VMEM((NB, GPC, 128), jnp.int32),
        pltpu.VMEM((NB, CHUNK, EMB), jnp.float32),
        pltpu.VMEM((2, EMB), jnp.float32),
        pltpu.SemaphoreType.DMA,
        pltpu.SemaphoreType.DMA,
        pltpu.SemaphoreType.DMA,
        pltpu.SemaphoreType.DMA,
        pltpu.SemaphoreType.DMA,
        pltpu.SemaphoreType.DMA,
        pltpu.SemaphoreType.DMA,
        pltpu.SemaphoreType.DMA,
        pltpu.SemaphoreType.DMA,
    ],
    compiler_params=pltpu.CompilerParams(needs_layout_passes=False),
)
def _emb_ln(ids_hbm, table_hbm, w_hbm, b_hbm, out_hbm,
            idx_v, rows_v, wb_v,
            sg0, sg1, sg2, so0, so1, so2, si0, si1, si2):
    sg = [sg0, sg1, sg2]
    so = [so0, so1, so2]
    si = [si0, si1, si2]
    wid = lax.axis_index("s") * NC + lax.axis_index("c")
    base_w = wid * TOK_PER_W

    pltpu.sync_copy(w_hbm, wb_v.at[0])
    pltpu.sync_copy(b_hbm, wb_v.at[1])
    w_vec = [wb_v[0, pl.ds(16 * j, 16)] for j in range(8)]
    b_vec = [wb_v[1, pl.ds(16 * j, 16)] for j in range(8)]

    def fire_idx(g, b):
        pltpu.async_copy(
            ids_hbm.at[pl.ds(wid * IDROWS_W + g * GPC, GPC)],
            idx_v.at[b],
            si[b],
        )

    def wait_idx(b):
        pltpu.make_async_copy(
            ids_hbm.at[pl.ds(0, GPC)],
            idx_v.at[b],
            si[b],
        ).wait()

    def fire_gathers(g, b):
        for u in range(GPC):
            pltpu.async_copy(
                table_hbm.at[idx_v.at[b].at[u]],
                rows_v.at[b].at[pl.ds(u * 128, 128)],
                sg[b],
            )

    def wait_gathers(b):
        for _ in range(GPC):
            pltpu.make_async_copy(
                table_hbm.at[idx_v.at[b].at[0]],
                rows_v.at[b].at[pl.ds(0, 128)],
                sg[b],
            ).wait()

    def fire_writeback(g, b):
        pltpu.async_copy(
            rows_v.at[b],
            out_hbm.at[pl.ds(base_w + g * CHUNK, CHUNK)],
            so[b],
        )

    def wait_writeback(b):
        pltpu.make_async_copy(
            rows_v.at[b],
            out_hbm.at[pl.ds(base_w, CHUNK)],
            so[b],
        ).wait()

    def compute_chunk(b):
        rbuf = rows_v.at[b]
        UNROLL = 8

        def tok(t, _):
            for u in range(UNROLL):
                row = t * UNROLL + u
                xs = [rbuf[row, pl.ds(16 * j, 16)] for j in range(8)]
                s1 = ((xs[0] + xs[1]) + (xs[2] + xs[3])) + (
                    (xs[4] + xs[5]) + (xs[6] + xs[7]))
                sq = [x * x for x in xs]
                s2 = ((sq[0] + sq[1]) + (sq[2] + sq[3])) + (
                    (sq[4] + sq[5]) + (sq[6] + sq[7]))
                tot = plsc.cumsum(s1)[15]
                tot2 = plsc.cumsum(s2)[15]
                mean = tot * (1.0 / EMB)
                var = tot2 * (1.0 / EMB) - mean * mean
                x = var + LN_EPS
                i = lax.bitcast_convert_type(x, jnp.int32)
                i = jnp.int32(0x5F3759DF) - (i >> 1)
                y = lax.bitcast_convert_type(i, jnp.float32)
                for _ in range(2):
                    y = y * (1.5 - 0.5 * x * y * y)
                for j in range(8):
                    rbuf[row, pl.ds(16 * j, 16)] = (
                        (xs[j] - mean) * (y * w_vec[j]) + b_vec[j]
                    )
            return 0

        lax.fori_loop(0, CHUNK // UNROLL, tok, 0)

    def step(g, b, fire, wait_out, fire_ids=True):
        wait_gathers(b)
        if fire_ids:
            fire_idx(g + 3, b)
        compute_chunk(b)
        fire_writeback(g, b)
        if fire:
            bn = (b + 2) % NB
            wait_idx(bn)
            if wait_out:
                wait_writeback(bn)
            fire_gathers(g + 2, bn)

    pltpu.sync_copy(ids_hbm.at[pl.ds(wid * IDROWS_W, GPC)], idx_v.at[0])
    pltpu.sync_copy(ids_hbm.at[pl.ds(wid * IDROWS_W + GPC, GPC)], idx_v.at[1])
    fire_idx(2, 2)
    fire_gathers(0, 0)
    fire_gathers(1, 1)
    step(0, 0, fire=True, wait_out=False)

    def steady(k, _):
        for j in range(NB):
            g = 1 + k * NB + j
            step(g, (1 + j) % NB, fire=True, wait_out=True)
        return 0

    lax.fori_loop(0, (NCHUNK - 4) // NB, steady, 0)

    step(NCHUNK - 3, (NCHUNK - 3) % NB, fire=True, wait_out=True, fire_ids=False)
    step(NCHUNK - 2, (NCHUNK - 2) % NB, fire=False, wait_out=False, fire_ids=False)
    step(NCHUNK - 1, (NCHUNK - 1) % NB, fire=False, wait_out=False, fire_ids=False)
    for b in range(NB):
        wait_writeback(b)


def kernel(input_ids, table, ln_weight, ln_bias):
    ids = input_ids.reshape(-1).astype(jnp.int32).reshape(N_TOKENS // 128, 128)
    out = _emb_ln(ids, table, ln_weight, ln_bias)
    return out.reshape(input_ids.shape[0], input_ids.shape[1], EMB)
